# Initial kernel scaffold; baseline (speedup 1.0000x reference)
import jax
import jax.numpy as jnp
from jax.experimental import pallas as pl


def kernel(x, edge_index, edge_values, weight):
    raise NotImplementedError("write your pallas kernel here")


# speedup vs baseline: 5.4023x; 5.4023x over previous
import functools

import jax
import jax.numpy as jnp
from jax import lax
from jax.experimental import pallas as pl
from jax.experimental.pallas import tpu as pltpu
from jax.experimental.pallas import tpu_sc as plsc


def _matmul_body(x_ref, w_ref, o_ref):
    o_ref[...] = jnp.dot(x_ref[...], w_ref[...],
                         preferred_element_type=jnp.float32)


def _matmul(x, w):
    m, k = x.shape
    n = w.shape[1]
    bm = 1000
    return pl.pallas_call(
        _matmul_body,
        grid=(m // bm,),
        in_specs=[
            pl.BlockSpec((bm, k), lambda i: (i, 0)),
            pl.BlockSpec((k, n), lambda i: (0, 0)),
        ],
        out_specs=pl.BlockSpec((bm, n), lambda i: (i, 0)),
        out_shape=jax.ShapeDtypeStruct((m, n), jnp.float32),
    )(x, w)


def _combine_body(p_ref, o_ref):
    o_ref[...] = jnp.maximum(p_ref[0] + p_ref[1], 0.0)


def _combine(partials):
    _, m, n = partials.shape
    bm = 1000
    return pl.pallas_call(
        _combine_body,
        grid=(m // bm,),
        in_specs=[pl.BlockSpec((2, bm, n), lambda i: (0, i, 0))],
        out_specs=pl.BlockSpec((bm, n), lambda i: (i, 0)),
        out_shape=jax.ShapeDtypeStruct((m, n), jnp.float32),
    )(partials)


_C = 128
_NW = 32


def _make_scatter(m, n, e):
    nchunks = e // _C
    assert nchunks * _C == e
    rows_per_tile = (m // 16) // 8 * 8
    rows_rem = m - rows_per_tile * 16
    assert rows_per_tile * 5 >= rows_per_tile + rows_rem
    mesh = plsc.VectorSubcoreMesh(core_axis_name="c", subcore_axis_name="s")

    @functools.partial(
        pl.kernel,
        out_type=jax.ShapeDtypeStruct((2, m, n), jnp.float32),
        mesh=mesh,
        scratch_types=[
            pltpu.VMEM_SHARED((m, n), jnp.float32),
            pltpu.VMEM((_C,), jnp.int32),
            pltpu.VMEM((_C,), jnp.int32),
            pltpu.VMEM((_C,), jnp.float32),
            pltpu.VMEM((_C, n), jnp.float32),
            pltpu.SemaphoreType.DMA,
        ],
    )
    def scatter(presup, src, dst, ev, out, acc, src_v, dst_v, ev_v, rows_v,
                sem):
        c = lax.axis_index("c")
        s = lax.axis_index("s")
        w = c * 16 + s

        zero16 = jnp.zeros((16,), jnp.float32)

        def zrow(i, carry):
            for j in range(n // 16):
                rows_v[i, pl.ds(j * 16, 16)] = zero16
            return carry

        lax.fori_loop(0, _C, zrow, 0)

        base = s * rows_per_tile
        zchunks = -(-(rows_per_tile + rows_rem) // _C)
        for kk in range(zchunks):
            pltpu.sync_copy(rows_v, acc.at[pl.ds(base + kk * _C, _C)])
        plsc.subcore_barrier()

        my_chunks = (nchunks - w + _NW - 1) // _NW

        def body(i, carry):
            b = (w + i * _NW) * _C
            pltpu.sync_copy(src.at[pl.ds(b, _C)], src_v)
            pltpu.sync_copy(dst.at[pl.ds(b, _C)], dst_v)
            pltpu.sync_copy(ev.at[pl.ds(b, _C)], ev_v)
            pltpu.async_copy(presup.at[src_v], rows_v, sem).wait()

            def scale(g, carry2):
                evv = ev_v[pl.ds(g * 16, 16)]
                for l in range(16):
                    sv = evv[l]
                    r = g * 16 + l
                    for j in range(n // 16):
                        rows_v[r, pl.ds(j * 16, 16)] = (
                            rows_v[r, pl.ds(j * 16, 16)] * sv)
                return carry2

            lax.fori_loop(0, _C // 16, scale, 0)
            pltpu.sync_copy(rows_v, acc.at[dst_v], add=True)
            return carry

        lax.fori_loop(0, my_chunks, body, 0)

        plsc.subcore_barrier()
        pltpu.sync_copy(acc.at[pl.ds(base, rows_per_tile)],
                        out.at[c, pl.ds(base, rows_per_tile)])
        if rows_rem:
            @pl.when(s == 15)
            def _():
                pltpu.sync_copy(
                    acc.at[pl.ds(16 * rows_per_tile, rows_rem)],
                    out.at[c, pl.ds(16 * rows_per_tile, rows_rem)])

    return scatter


def kernel(x, edge_index, edge_values, weight):
    m, _ = x.shape
    n = weight.shape[1]
    e = edge_values.shape[0]
    presup = _matmul(x, weight)
    src = edge_index[0].astype(jnp.int32)
    dst = edge_index[1].astype(jnp.int32)
    partials = _make_scatter(m, n, e)(presup, src, dst, edge_values)
    return _combine(partials)


# speedup vs baseline: 10.1339x; 1.8759x over previous
import functools

import jax
import jax.numpy as jnp
from jax import lax
from jax.experimental import pallas as pl
from jax.experimental.pallas import tpu as pltpu
from jax.experimental.pallas import tpu_sc as plsc


def _matmul_body(x_ref, w_ref, o_ref):
    o_ref[...] = jnp.dot(x_ref[...], w_ref[...],
                         preferred_element_type=jnp.float32)


def _matmul(x, w):
    m, k = x.shape
    n = w.shape[1]
    bm = 1000
    return pl.pallas_call(
        _matmul_body,
        grid=(m // bm,),
        in_specs=[
            pl.BlockSpec((bm, k), lambda i: (i, 0)),
            pl.BlockSpec((k, n), lambda i: (0, 0)),
        ],
        out_specs=pl.BlockSpec((bm, n), lambda i: (i, 0)),
        out_shape=jax.ShapeDtypeStruct((m, n), jnp.float32),
    )(x, w)


def _combine_body(p_ref, o_ref):
    o_ref[...] = jnp.maximum(p_ref[0] + p_ref[1], 0.0)


def _combine(partials):
    _, m, n = partials.shape
    bm = 1000
    return pl.pallas_call(
        _combine_body,
        grid=(m // bm,),
        in_specs=[pl.BlockSpec((2, bm, n), lambda i: (0, i, 0))],
        out_specs=pl.BlockSpec((bm, n), lambda i: (i, 0)),
        out_shape=jax.ShapeDtypeStruct((m, n), jnp.float32),
    )(partials)


_C = 100
_NW = 32
_NJ = 100


def _make_scatter(m, n):
    rows_per_tile = (m // 16) // 8 * 8
    rows_rem = m - rows_per_tile * 16
    cover = rows_per_tile + rows_rem
    mesh = plsc.VectorSubcoreMesh(core_axis_name="c", subcore_axis_name="s")

    @functools.partial(
        pl.kernel,
        out_type=jax.ShapeDtypeStruct((2, m, n), jnp.float32),
        mesh=mesh,
        scratch_types=[
            pltpu.VMEM_SHARED((m, n), jnp.float32),
            pltpu.VMEM((1, _C), jnp.int32),
            pltpu.VMEM((1, _C), jnp.int32),
            pltpu.VMEM((1, _C), jnp.int32),
            pltpu.VMEM((1, _C), jnp.int32),
            pltpu.VMEM((1, _C), jnp.int32),
            pltpu.VMEM((1, _C), jnp.int32),
            pltpu.VMEM((1, _C), jnp.float32),
            pltpu.VMEM((1, _C), jnp.float32),
            pltpu.VMEM((1, _C), jnp.float32),
            pltpu.VMEM((_C, n), jnp.float32),
            pltpu.VMEM((_C, n), jnp.float32),
            pltpu.VMEM((_C, n), jnp.float32),
            pltpu.SemaphoreType.DMA,
            pltpu.SemaphoreType.DMA,
            pltpu.SemaphoreType.DMA,
            pltpu.SemaphoreType.DMA,
            pltpu.SemaphoreType.DMA,
            pltpu.SemaphoreType.DMA,
            pltpu.SemaphoreType.DMA,
            pltpu.SemaphoreType.DMA,
            pltpu.SemaphoreType.DMA,
        ],
    )
    def scatter(src, dst, ev, presup, out, acc,
                sb0, sb1, sb2, db0, db1, db2, eb0, eb1, eb2,
                r0, r1, r2, g0, g1, g2, s0, s1, s2, i0, i1, i2):
        ib_src = (sb0, sb1, sb2)
        ib_dst = (db0, db1, db2)
        ib_ev = (eb0, eb1, eb2)
        rows = (r0, r1, r2)
        gsem = (g0, g1, g2)
        ssem = (s0, s1, s2)
        isem = (i0, i1, i2)
        c = lax.axis_index("c")
        s = lax.axis_index("s")
        w = c * 16 + s
        row0 = w * _NJ

        zero16 = jnp.zeros((16,), jnp.float32)

        def zrow(i, carry):
            for j in range(n // 16):
                r0[i, pl.ds(j * 16, 16)] = zero16
            return carry

        lax.fori_loop(0, _C, zrow, 0)
        base = s * rows_per_tile
        zfull = cover // _C
        zrem = cover - zfull * _C
        for kk in range(zfull):
            pltpu.sync_copy(r0, acc.at[pl.ds(base + kk * _C, _C)])
        if zrem:
            pltpu.sync_copy(r0.at[pl.ds(0, zrem)],
                            acc.at[pl.ds(base + zfull * _C, zrem)])
        plsc.subcore_barrier()

        def start_idx(j, k):
            pltpu.async_copy(src.at[row0 + j], ib_src[k], isem[k])
            pltpu.async_copy(dst.at[row0 + j], ib_dst[k], isem[k])
            pltpu.async_copy(ev.at[row0 + j], ib_ev[k], isem[k])

        def wait_idx(k):
            pltpu.make_async_copy(src.at[0], ib_src[k], isem[k]).wait()
            pltpu.make_async_copy(dst.at[0], ib_dst[k], isem[k]).wait()
            pltpu.make_async_copy(ev.at[0], ib_ev[k], isem[k]).wait()

        def start_gather(k, b):
            pltpu.async_copy(presup.at[ib_src[k].at[0]], rows[b], gsem[b])

        def wait_gather(b):
            pltpu.make_async_copy(presup.at[ib_src[0].at[0]], rows[b],
                                  gsem[b]).wait()

        def start_scatter(k, b):
            pltpu.async_copy(rows[b], acc.at[ib_dst[k].at[0]], ssem[b],
                             add=True)

        def wait_scatter(b):
            pltpu.make_async_copy(rows[b], acc.at[ib_dst[0].at[0]],
                                  ssem[b]).wait()

        def scale(k, b):
            buf = rows[b]
            evb = ib_ev[k]

            def inner(gg, carry):
                evv = evb[0, pl.ds(gg * 16, 16)]
                for l in range(16):
                    sv = evv[l]
                    r = gg * 16 + l
                    for q in range(n // 16):
                        buf[r, pl.ds(q * 16, 16)] = (
                            buf[r, pl.ds(q * 16, 16)] * sv)
                return carry

            lax.fori_loop(0, (_C // 16), inner, 0)
            evv = evb[0, pl.ds(_C - 16, 16)]
            for l in range(16 - (_C - _C // 16 * 16), 16):
                sv = evv[l]
                r = _C - 16 + l
                for q in range(n // 16):
                    buf[r, pl.ds(q * 16, 16)] = (
                        buf[r, pl.ds(q * 16, 16)] * sv)

        def chunk(j, t, first, idx_guard):
            bnext = (t + 1) % 3
            bn = (t + 2) % 3
            wait_gather(t)
            wait_idx(bnext)
            start_gather(bnext, bnext)
            scale(t, t)
            start_scatter(t, t)
            if not first:
                wait_scatter(bn)
            if idx_guard is None:
                start_idx(j + 2, bn)
            elif idx_guard:
                @pl.when(j <= _NJ - 3)
                def _():
                    start_idx(j + 2, bn)

        start_idx(0, 0)
        start_idx(1, 1)
        wait_idx(0)
        start_gather(0, 0)
        chunk(0, 0, True, None)
        chunk(1, 1, False, None)
        chunk(2, 2, False, None)

        def group(g, carry):
            for t in range(3):
                chunk(g * 3 + t, t, False, True)
            return carry

        lax.fori_loop(1, _NJ // 3, group, 0)

        wait_gather(0)
        scale(0, 0)
        start_scatter(0, 0)
        wait_scatter(2)
        wait_scatter(0)

        plsc.subcore_barrier()
        pltpu.sync_copy(acc.at[pl.ds(base, rows_per_tile)],
                        out.at[c, pl.ds(base, rows_per_tile)])
        if rows_rem:
            @pl.when(s == 15)
            def _():
                pltpu.sync_copy(
                    acc.at[pl.ds(16 * rows_per_tile, rows_rem)],
                    out.at[c, pl.ds(16 * rows_per_tile, rows_rem)])

    return scatter


def kernel(x, edge_index, edge_values, weight):
    m, _ = x.shape
    n = weight.shape[1]
    e = edge_values.shape[0]
    assert e == _NW * _NJ * _C
    presup = _matmul(x, weight)
    src = edge_index[0].astype(jnp.int32).reshape(_NW * _NJ, 1, _C)
    dst = edge_index[1].astype(jnp.int32).reshape(_NW * _NJ, 1, _C)
    ev = edge_values.reshape(_NW * _NJ, 1, _C)
    partials = _make_scatter(m, n)(src, dst, ev, presup)
    return _combine(partials)
